# Initial kernel scaffold; baseline (speedup 1.0000x reference)
import jax
import jax.numpy as jnp
from jax.experimental import pallas as pl


def kernel(word, context, negative_samples, word_embed, ctx_embed):
    raise NotImplementedError("write your pallas kernel here")


# speedup vs baseline: 2.5236x; 2.5236x over previous
import functools

import jax
import jax.numpy as jnp
from jax import lax
from jax.experimental import pallas as pl
from jax.experimental.pallas import tpu as pltpu
from jax.experimental.pallas import tpu_sc as plsc

VOCAB = 1000000
EMBED = 64
BATCH = 16384
NEG = 10

NUM_CORES = 2
NUM_SUBCORES = 16
NUM_WORKERS = NUM_CORES * NUM_SUBCORES
ITEMS_PER_WORKER = BATCH // NUM_WORKERS
SUB = 128
NSUB = ITEMS_PER_WORKER // SUB
GROUPS = SUB // 16


def _sc_scores_kernel(word_hbm, ctx_hbm, negt_hbm, wtab_hbm, ctab_hbm,
                      pos_out, neg_out,
                      widx, cidx, nidx, xrows, yrows, nrows,
                      poss, negs, sem):
    wid = lax.axis_index("s") * NUM_CORES + lax.axis_index("c")
    base_w = wid * ITEMS_PER_WORKER

    lane = lax.iota(jnp.int32, 16)

    for c in range(NSUB):
        base_c = base_w + c * SUB
        pltpu.sync_copy(word_hbm.at[pl.ds(base_c, SUB)], widx)
        pltpu.sync_copy(ctx_hbm.at[pl.ds(base_c, SUB)], cidx)
        for k in range(NEG):
            pltpu.sync_copy(negt_hbm.at[k, pl.ds(base_c, SUB)], nidx.at[k])
        cps = [pltpu.async_copy(wtab_hbm.at[widx], xrows, sem),
               pltpu.async_copy(ctab_hbm.at[cidx], yrows, sem)]
        for k in range(NEG):
            cps.append(pltpu.async_copy(ctab_hbm.at[nidx.at[k]], nrows.at[k], sem))
        for cp in cps:
            cp.wait()

        def group_body(g, _):
            rows = g * 16 + lane

            def d_body(d, accs):
                col = jnp.full((16,), d, jnp.int32)
                xv = plsc.load_gather(xrows, [rows, col])
                yv = plsc.load_gather(yrows, [rows, col])
                acc_p = accs[0] + xv * yv
                new = [acc_p]
                for k in range(NEG):
                    nv = plsc.load_gather(nrows, [jnp.full((16,), k, jnp.int32), rows, col])
                    new.append(accs[1 + k] + xv * nv)
                return new

            accs = [jnp.zeros((16,), jnp.float32) for _ in range(1 + NEG)]
            accs = lax.fori_loop(0, EMBED, d_body, accs)
            start = c * SUB + g * 16
            poss[pl.ds(start, 16)] = accs[0]
            for k in range(NEG):
                plsc.store_scatter(negs, [lane * NEG + (start * NEG + k)], accs[1 + k])
            return 0

        lax.fori_loop(0, GROUPS, group_body, 0)

    pltpu.sync_copy(poss, pos_out.at[pl.ds(base_w, ITEMS_PER_WORKER)])
    pltpu.sync_copy(negs, neg_out.at[pl.ds(base_w * NEG, ITEMS_PER_WORKER * NEG)])


def _loss_body(pos_ref, neg_ref, out_ref):
    def logsig(z):
        return jnp.minimum(z, 0.0) - jnp.log1p(jnp.exp(-jnp.abs(z)))

    s = jnp.sum(logsig(pos_ref[...])) + jnp.sum(logsig(-neg_ref[...]))
    out_ref[...] = jnp.full((1, 1), -s / BATCH, jnp.float32)


def kernel(word, context, negative_samples, word_embed, ctx_embed):
    negt = negative_samples.T

    mesh = plsc.VectorSubcoreMesh(core_axis_name="c", subcore_axis_name="s")
    sc = functools.partial(
        pl.kernel,
        mesh=mesh,
        compiler_params=pltpu.CompilerParams(
            needs_layout_passes=False, use_tc_tiling_on_sc=False),
        out_type=[jax.ShapeDtypeStruct((BATCH,), jnp.float32),
                  jax.ShapeDtypeStruct((BATCH * NEG,), jnp.float32)],
        scratch_types=[
            pltpu.VMEM((SUB,), jnp.int32),
            pltpu.VMEM((SUB,), jnp.int32),
            pltpu.VMEM((NEG, SUB), jnp.int32),
            pltpu.VMEM((SUB, EMBED), jnp.float32),
            pltpu.VMEM((SUB, EMBED), jnp.float32),
            pltpu.VMEM((NEG, SUB, EMBED), jnp.float32),
            pltpu.VMEM((ITEMS_PER_WORKER,), jnp.float32),
            pltpu.VMEM((ITEMS_PER_WORKER * NEG,), jnp.float32),
            pltpu.SemaphoreType.DMA,
        ],
    )(_sc_scores_kernel)
    pos_sc, neg_sc = sc(word, context, negt, word_embed, ctx_embed)

    loss2d = pl.pallas_call(
        _loss_body,
        out_shape=jax.ShapeDtypeStruct((1, 1), jnp.float32),
    )(pos_sc.reshape(BATCH // 128, 128), neg_sc.reshape(BATCH * NEG // 128, 128))
    return loss2d[0, 0]


# speedup vs baseline: 2.8511x; 1.1298x over previous
import functools

import jax
import jax.numpy as jnp
from jax import lax
from jax.experimental import pallas as pl
from jax.experimental.pallas import tpu as pltpu
from jax.experimental.pallas import tpu_sc as plsc

VOCAB = 1000000
EMBED = 64
BATCH = 16384
NEG = 10

NUM_CORES = 2
NUM_SUBCORES = 16
NUM_WORKERS = NUM_CORES * NUM_SUBCORES
ITEMS_PER_WORKER = BATCH // NUM_WORKERS
SUB = 128
NSUB = ITEMS_PER_WORKER // SUB
NCHUNK = EMBED // 16

POS_PART = BATCH * 16
TOTAL_PART = (BATCH + BATCH * NEG) * 16
PART_ROWS = TOTAL_PART // 128


def _sc_scores_kernel(word_hbm, ctx_hbm, negt_hbm, wtab_hbm, ctab_hbm,
                      part_out,
                      widx, cidx, nidx, xrows, yrows, nrows,
                      ppart, npart, sem):
    wid = lax.axis_index("s") * NUM_CORES + lax.axis_index("c")
    base_w = wid * ITEMS_PER_WORKER

    for c in range(NSUB):
        base_c = base_w + c * SUB
        pltpu.sync_copy(word_hbm.at[pl.ds(base_c, SUB)], widx)
        pltpu.sync_copy(ctx_hbm.at[pl.ds(base_c, SUB)], cidx)
        for k in range(NEG):
            pltpu.sync_copy(negt_hbm.at[k, pl.ds(base_c, SUB)], nidx.at[k])
        cps = [pltpu.async_copy(wtab_hbm.at[widx], xrows, sem),
               pltpu.async_copy(ctab_hbm.at[cidx], yrows, sem)]
        for k in range(NEG):
            cps.append(pltpu.async_copy(ctab_hbm.at[nidx.at[k]], nrows.at[k], sem))
        for cp in cps:
            cp.wait()

        def item_body(i, _):
            xs = [xrows[i, pl.ds(j * 16, 16)] for j in range(NCHUNK)]
            acc = xs[0] * yrows[i, pl.ds(0, 16)]
            for j in range(1, NCHUNK):
                acc = acc + xs[j] * yrows[i, pl.ds(j * 16, 16)]
            ppart[pl.ds(i * 16, 16)] = acc
            for k in range(NEG):
                acc = xs[0] * nrows[k, i, pl.ds(0, 16)]
                for j in range(1, NCHUNK):
                    acc = acc + xs[j] * nrows[k, i, pl.ds(j * 16, 16)]
                npart[pl.ds((i * NEG + k) * 16, 16)] = acc
            return 0

        lax.fori_loop(0, SUB, item_body, 0)

        pltpu.sync_copy(ppart, part_out.at[pl.ds(base_c * 16, SUB * 16)])
        pltpu.sync_copy(
            npart,
            part_out.at[pl.ds(POS_PART + base_c * NEG * 16, SUB * NEG * 16)])


def _loss_body(part_ref, out_ref):
    x = part_ref[...]
    r = lax.broadcasted_iota(jnp.int32, (128, 8), 0) // 16
    g = lax.broadcasted_iota(jnp.int32, (128, 8), 1)
    gmat = (r == g).astype(jnp.float32)
    s = jax.lax.dot_general(x, gmat, (((1,), (0,)), ((), ())),
                            preferred_element_type=jnp.float32)
    row = lax.broadcasted_iota(jnp.int32, (PART_ROWS, 8), 0)
    z = jnp.where(row < (BATCH * 16) // 128, s, -s)
    l = jnp.minimum(z, 0.0) - jnp.log1p(jnp.exp(-jnp.abs(z)))
    out_ref[...] = jnp.full((1, 1), -jnp.sum(l) / BATCH, jnp.float32)


def kernel(word, context, negative_samples, word_embed, ctx_embed):
    negt = negative_samples.T

    mesh = plsc.VectorSubcoreMesh(core_axis_name="c", subcore_axis_name="s")
    sc = functools.partial(
        pl.kernel,
        mesh=mesh,
        compiler_params=pltpu.CompilerParams(
            needs_layout_passes=False, use_tc_tiling_on_sc=False),
        out_type=jax.ShapeDtypeStruct((TOTAL_PART,), jnp.float32),
        scratch_types=[
            pltpu.VMEM((SUB,), jnp.int32),
            pltpu.VMEM((SUB,), jnp.int32),
            pltpu.VMEM((NEG, SUB), jnp.int32),
            pltpu.VMEM((SUB, EMBED), jnp.float32),
            pltpu.VMEM((SUB, EMBED), jnp.float32),
            pltpu.VMEM((NEG, SUB, EMBED), jnp.float32),
            pltpu.VMEM((SUB * 16,), jnp.float32),
            pltpu.VMEM((SUB * NEG * 16,), jnp.float32),
            pltpu.SemaphoreType.DMA,
        ],
    )(_sc_scores_kernel)
    part = sc(word, context, negt, word_embed, ctx_embed)

    loss2d = pl.pallas_call(
        _loss_body,
        out_shape=jax.ShapeDtypeStruct((1, 1), jnp.float32),
    )(part.reshape(PART_ROWS, 128))
    return loss2d[0, 0]
